# baseline (speedup 1.0000x reference)
import functools

import jax
import jax.numpy as jnp
from jax import lax
from jax.experimental import pallas as pl
from jax.experimental.pallas import tpu as pltpu
from jax.experimental.pallas import tpu_sc as plsc

VOCAB = 1000000
BATCH = 16384
DIM = 64

_NC = 2
_NS = 16
_NW = _NC * _NS
_BPW = BATCH // _NW

_mesh = plsc.VectorSubcoreMesh(core_axis_name="c", subcore_axis_name="s")


@functools.partial(
    pl.kernel,
    mesh=_mesh,
    out_type=[
        jax.ShapeDtypeStruct((BATCH, DIM), jnp.float32),
        jax.ShapeDtypeStruct((BATCH, DIM), jnp.float32),
    ],
    scratch_types=[
        pltpu.VMEM((_BPW,), jnp.int32),
        pltpu.SemaphoreType.DMA,
    ],
)
def _sc_gather(x0_hbm, x1_hbm, idx_hbm, g0_hbm, g1_hbm, idx_v, sem):
    wid = lax.axis_index("s") * _NC + lax.axis_index("c")
    base = wid * _BPW
    pltpu.sync_copy(idx_hbm.at[pl.ds(base, _BPW)], idx_v)

    def issue_chunk(c, carry):
        off = c * 16
        vec = idx_v[pl.ds(off, 16)]
        for k in range(16):
            i = vec[k]
            pltpu.make_async_copy(
                x0_hbm.at[pl.ds(i, 1), :],
                g0_hbm.at[pl.ds(base + off + k, 1), :],
                sem,
            ).start()
            pltpu.make_async_copy(
                x1_hbm.at[pl.ds(i, 1), :],
                g1_hbm.at[pl.ds(base + off + k, 1), :],
                sem,
            ).start()
        return carry

    lax.fori_loop(0, _BPW // 16, issue_chunk, 0)
    pltpu.make_async_copy(
        x0_hbm.at[pl.ds(0, _BPW), :],
        g0_hbm.at[pl.ds(base, _BPW), :],
        sem,
    ).wait()
    pltpu.make_async_copy(
        x1_hbm.at[pl.ds(0, _BPW), :],
        g1_hbm.at[pl.ds(base, _BPW), :],
        sem,
    ).wait()


_BM = 1024


def _mm_body(g0_ref, g1_ref, w0_ref, w1_ref, o_ref):
    o_ref[...] = (
        jnp.dot(g0_ref[...], w0_ref[...], preferred_element_type=jnp.float32)
        + jnp.dot(g1_ref[...], w1_ref[...], preferred_element_type=jnp.float32)
    )


_mm = pl.pallas_call(
    _mm_body,
    grid=(BATCH // _BM,),
    in_specs=[
        pl.BlockSpec((_BM, DIM), lambda i: (i, 0)),
        pl.BlockSpec((_BM, DIM), lambda i: (i, 0)),
        pl.BlockSpec((DIM, DIM), lambda i: (0, 0)),
        pl.BlockSpec((DIM, DIM), lambda i: (0, 0)),
    ],
    out_specs=pl.BlockSpec((_BM, DIM), lambda i: (i, 0)),
    out_shape=jax.ShapeDtypeStruct((BATCH, DIM), jnp.float32),
)


def kernel(x0, x1, W, indices):
    g0, g1 = _sc_gather(x0, x1, indices)
    return _mm(g0, g1, W[:DIM], W[DIM:])


# speedup vs baseline: 1.6956x; 1.6956x over previous
import functools

import jax
import jax.numpy as jnp
from jax import lax
from jax.experimental import pallas as pl
from jax.experimental.pallas import tpu as pltpu
from jax.experimental.pallas import tpu_sc as plsc

VOCAB = 1000000
BATCH = 16384
DIM = 64

_NC = 2
_NS = 16
_NW = _NC * _NS
_BPW = BATCH // _NW
_CH = 32
_NCHUNK = _BPW // _CH

_mesh = plsc.VectorSubcoreMesh(core_axis_name="c", subcore_axis_name="s")


@functools.partial(
    pl.kernel,
    mesh=_mesh,
    out_type=[
        jax.ShapeDtypeStruct((BATCH, DIM), jnp.float32),
        jax.ShapeDtypeStruct((BATCH, DIM), jnp.float32),
    ],
    scratch_types=[
        pltpu.VMEM((_BPW,), jnp.int32),
        pltpu.VMEM((_BPW, DIM), jnp.float32),
        pltpu.SemaphoreType.DMA,
    ],
)
def _sc_gather(x0_hbm, x1_hbm, idx_hbm, g0_hbm, g1_hbm, idx_v, rows_v, sem):
    wid = lax.axis_index("s") * _NC + lax.axis_index("c")
    base = wid * _BPW
    pltpu.sync_copy(idx_hbm.at[pl.ds(base, _BPW)], idx_v)

    def one_table(x_hbm, g_hbm):
        def issue_chunk(c, carry):
            off = c * 16
            vec = idx_v[pl.ds(off, 16)]
            for k in range(16):
                i = vec[k]
                pltpu.make_async_copy(
                    x_hbm.at[pl.ds(i, 1), :],
                    rows_v.at[pl.ds(off + k, 1), :],
                    sem,
                ).start()
            return carry

        lax.fori_loop(0, _BPW // 16, issue_chunk, 0)
        pltpu.make_async_copy(
            x_hbm.at[pl.ds(0, _BPW), :], rows_v, sem
        ).wait()
        pltpu.sync_copy(rows_v, g_hbm.at[pl.ds(base, _BPW), :])

    one_table(x0_hbm, g0_hbm)
    one_table(x1_hbm, g1_hbm)


_BM = 1024


def _mm_body(g0_ref, g1_ref, w0_ref, w1_ref, o_ref):
    o_ref[...] = (
        jnp.dot(g0_ref[...], w0_ref[...], preferred_element_type=jnp.float32)
        + jnp.dot(g1_ref[...], w1_ref[...], preferred_element_type=jnp.float32)
    )


_mm = pl.pallas_call(
    _mm_body,
    grid=(BATCH // _BM,),
    in_specs=[
        pl.BlockSpec((_BM, DIM), lambda i: (i, 0)),
        pl.BlockSpec((_BM, DIM), lambda i: (i, 0)),
        pl.BlockSpec((DIM, DIM), lambda i: (0, 0)),
        pl.BlockSpec((DIM, DIM), lambda i: (0, 0)),
    ],
    out_specs=pl.BlockSpec((_BM, DIM), lambda i: (i, 0)),
    out_shape=jax.ShapeDtypeStruct((BATCH, DIM), jnp.float32),
)


def kernel(x0, x1, W, indices):
    g0, g1 = _sc_gather(x0, x1, indices)
    return _mm(g0, g1, W[:DIM], W[DIM:])
